# baseline (speedup 1.0000x reference)
import functools

import jax
import jax.numpy as jnp
from jax.experimental import pallas as pl


def _gcn_body(x_ref, a_ref, w1_ref, b1_ref, w2_ref, b2_ref, o_ref):
    slope = jnp.float32(0.01)
    a = a_ref[...]
    s = jnp.dot(a, x_ref[...].astype(jnp.bfloat16),
                preferred_element_type=jnp.float32)
    h = jnp.dot(s.astype(jnp.bfloat16), w1_ref[...],
                preferred_element_type=jnp.float32) + b1_ref[...]
    h = jnp.where(h >= 0, h, slope * h)
    s2 = jnp.dot(a, h.astype(jnp.bfloat16), preferred_element_type=jnp.float32)
    o = jnp.dot(s2.astype(jnp.bfloat16), w2_ref[...],
                preferred_element_type=jnp.float32) + b2_ref[...]
    o_ref[...] = jnp.where(o >= 0, o, slope * o)


@jax.jit
def _gcn_block(Xc, A, W1e, b1t, W2e, b2t):
    N = A.shape[0]
    kin_blk = W1e.shape[0]
    kout_blk = W2e.shape[1]
    steps = Xc.shape[0] // N
    return pl.pallas_call(
        _gcn_body,
        grid=(steps,),
        in_specs=[
            pl.BlockSpec((N, kin_blk), lambda g: (g, 0)),
            pl.BlockSpec((N, N), lambda g: (0, 0)),
            pl.BlockSpec((kin_blk, W1e.shape[1]), lambda g: (0, 0)),
            pl.BlockSpec((1, W1e.shape[1]), lambda g: (0, 0)),
            pl.BlockSpec((W2e.shape[0], kout_blk), lambda g: (0, 0)),
            pl.BlockSpec((1, kout_blk), lambda g: (0, 0)),
        ],
        out_specs=pl.BlockSpec((N, kout_blk), lambda g: (g, 0)),
        out_shape=jax.ShapeDtypeStruct((steps * N, kout_blk), jnp.float32),
    )(Xc, A, W1e, b1t, W2e, b2t)


def kernel(X, A, W1, b1, W2, b2):
    B, N, T, F_in = X.shape
    F_sp = W1.shape[1]

    Xc = X.reshape(B * N, T * F_in)

    eye = jnp.eye(T, dtype=jnp.float32)
    W1e = jnp.kron(eye, W1).astype(jnp.bfloat16)
    W2e = jnp.kron(eye, W2).astype(jnp.bfloat16)
    b1t = jnp.tile(b1, T)[None, :]
    b2t = jnp.tile(b2, T)[None, :]

    out = _gcn_block(Xc, A.astype(jnp.bfloat16), W1e, b1t, W2e, b2t)
    return out.reshape(B, N, T, F_sp)


# speedup vs baseline: 1.0167x; 1.0167x over previous
import functools

import jax
import jax.numpy as jnp
from jax.experimental import pallas as pl


def _gcn_body(x_ref, a_ref, w1_ref, b1_ref, w2_ref, b2_ref, o_ref):
    slope = jnp.float32(0.01)
    a = a_ref[...]
    w1 = w1_ref[...]
    w2 = w2_ref[...]
    b1 = b1_ref[...]
    b2 = b2_ref[...]
    nn = a.shape[0]
    n_chains = x_ref.shape[0] // nn
    for c in range(n_chains):
        x = x_ref[c * nn:(c + 1) * nn, :].astype(jnp.bfloat16)
        s = jnp.dot(a, x, preferred_element_type=jnp.float32)
        h = jnp.dot(s.astype(jnp.bfloat16), w1,
                    preferred_element_type=jnp.float32) + b1
        h = jnp.where(h >= 0, h, slope * h)
        s2 = jnp.dot(a, h.astype(jnp.bfloat16),
                     preferred_element_type=jnp.float32)
        o = jnp.dot(s2.astype(jnp.bfloat16), w2,
                    preferred_element_type=jnp.float32) + b2
        o_ref[c * nn:(c + 1) * nn, :] = jnp.where(o >= 0, o, slope * o)


@functools.partial(jax.jit, static_argnames=("chains",))
def _gcn_block(Xc, A, W1e, b1t, W2e, b2t, chains):
    N = A.shape[0]
    kin_blk = W1e.shape[0]
    kout_blk = W2e.shape[1]
    steps = Xc.shape[0] // (N * chains)
    return pl.pallas_call(
        _gcn_body,
        grid=(steps,),
        in_specs=[
            pl.BlockSpec((chains * N, kin_blk), lambda g: (g, 0)),
            pl.BlockSpec((N, N), lambda g: (0, 0)),
            pl.BlockSpec((kin_blk, W1e.shape[1]), lambda g: (0, 0)),
            pl.BlockSpec((1, W1e.shape[1]), lambda g: (0, 0)),
            pl.BlockSpec((W2e.shape[0], kout_blk), lambda g: (0, 0)),
            pl.BlockSpec((1, kout_blk), lambda g: (0, 0)),
        ],
        out_specs=pl.BlockSpec((chains * N, kout_blk), lambda g: (g, 0)),
        out_shape=jax.ShapeDtypeStruct((steps * chains * N, kout_blk), jnp.float32),
    )(Xc, A, W1e, b1t, W2e, b2t)


def kernel(X, A, W1, b1, W2, b2):
    B, N, T, F_in = X.shape
    F_sp = W1.shape[1]

    Xc = X.reshape(B * N, T * F_in)

    eye = jnp.eye(T, dtype=jnp.float32)
    W1e = jnp.kron(eye, W1).astype(jnp.bfloat16)
    W2e = jnp.kron(eye, W2).astype(jnp.bfloat16)
    b1t = jnp.tile(b1, T)[None, :]
    b2t = jnp.tile(b2, T)[None, :]

    out = _gcn_block(Xc, A.astype(jnp.bfloat16), W1e, b1t, W2e, b2t, chains=2)
    return out.reshape(B, N, T, F_sp)
